# baseline (speedup 1.0000x reference)
import functools

import jax
import jax.numpy as jnp
from jax import lax
from jax.experimental import pallas as pl
from jax.experimental.pallas import tpu as pltpu
from jax.experimental.pallas import tpu_sc as plsc

S1 = 4
S2 = 12
D = 64
NSEL = 4
NC, NS = 2, 16
NW = NC * NS
BLKB = 128
W2 = S2 * D


def _build_sc_gather(B, ne, nr):
    bpw = B // NW
    mesh = plsc.VectorSubcoreMesh(
        core_axis_name="c", subcore_axis_name="s",
        num_cores=NC, num_subcores=NS)

    @functools.partial(
        pl.kernel,
        mesh=mesh,
        compiler_params=pltpu.CompilerParams(
            needs_layout_passes=False, use_tc_tiling_on_sc=False),
        out_type=[
            jax.ShapeDtypeStruct((B, D), jnp.bfloat16),
            jax.ShapeDtypeStruct((B, D), jnp.bfloat16),
            jax.ShapeDtypeStruct((B, D), jnp.bfloat16),
            jax.ShapeDtypeStruct((NSEL * B, D), jnp.bfloat16),
            jax.ShapeDtypeStruct((NSEL * B, D), jnp.bfloat16),
        ],
        scratch_types=[
            pltpu.VMEM((16, bpw), jnp.float32),
            pltpu.VMEM((16, bpw), jnp.int32),
            pltpu.VMEM((16, bpw), jnp.int32),
            pltpu.VMEM((3, bpw), jnp.int32),
            pltpu.VMEM((bpw,), jnp.int32),
            pltpu.VMEM((bpw,), jnp.int32),
            pltpu.VMEM((bpw,), jnp.int32),
            pltpu.VMEM((NSEL * bpw,), jnp.int32),
            pltpu.VMEM((NSEL * bpw,), jnp.int32),
            pltpu.VMEM((bpw, D), jnp.bfloat16),
            pltpu.VMEM((bpw, D), jnp.bfloat16),
            pltpu.VMEM((bpw, D), jnp.bfloat16),
            pltpu.VMEM((NSEL * bpw, D), jnp.bfloat16),
            pltpu.VMEM((NSEL * bpw, D), jnp.bfloat16),
            pltpu.SemaphoreType.DMA,
        ],
    )
    def sc_gather(ent_h, rel_h, timeT_h, nhT_h, ntT_h, spoT_h,
                  es_h, ep_h, eo_h, gh_h, gt_h,
                  timeT_v, nhT_v, ntT_v, spoT_v,
                  sidx_v, pidx_v, oidx_v, hidx_v, tidx_v,
                  srow_v, prow_v, orow_v, hrow_v, trow_v, sem):
        wid = lax.axis_index("s") * NC + lax.axis_index("c")
        base = wid * bpw

        pltpu.sync_copy(timeT_h.at[:, pl.ds(base, bpw)], timeT_v)
        pltpu.sync_copy(nhT_h.at[:, pl.ds(base, bpw)], nhT_v)
        pltpu.sync_copy(ntT_h.at[:, pl.ds(base, bpw)], ntT_v)
        pltpu.sync_copy(spoT_h.at[:, pl.ds(base, bpw)], spoT_v)

        for g in range(bpw // 16):
            sl = pl.ds(g * 16, 16)
            lanes = jnp.arange(16, dtype=jnp.int32) + (g * 16)
            fv = (timeT_v[1, sl] + 2.0 * timeT_v[2, sl]
                  + 3.0 * timeT_v[3, sl])
            fi = fv.astype(jnp.int32)
            plsc.store_scatter(sidx_v, [lanes], spoT_v[0, sl])
            plsc.store_scatter(pidx_v, [lanes], spoT_v[1, sl])
            plsc.store_scatter(oidx_v, [lanes], spoT_v[2, sl])
            for k in range(NSEL):
                rowsel = NSEL * fi + k
                dst = lanes + (k * bpw)
                plsc.store_scatter(hidx_v, [dst],
                                   plsc.load_gather(nhT_v, [rowsel, lanes]))
                plsc.store_scatter(tidx_v, [dst],
                                   plsc.load_gather(ntT_v, [rowsel, lanes]))

        cps = pltpu.async_copy(ent_h.at[sidx_v], srow_v, sem)
        cpp = pltpu.async_copy(rel_h.at[pidx_v], prow_v, sem)
        cpo = pltpu.async_copy(ent_h.at[oidx_v], orow_v, sem)
        cph = pltpu.async_copy(ent_h.at[hidx_v], hrow_v, sem)
        cpt = pltpu.async_copy(ent_h.at[tidx_v], trow_v, sem)
        cps.wait()
        cpp.wait()
        cpo.wait()
        cph.wait()
        cpt.wait()

        pltpu.sync_copy(srow_v, es_h.at[pl.ds(base, bpw)])
        pltpu.sync_copy(prow_v, ep_h.at[pl.ds(base, bpw)])
        pltpu.sync_copy(orow_v, eo_h.at[pl.ds(base, bpw)])
        for k in range(NSEL):
            pltpu.sync_copy(hrow_v.at[pl.ds(k * bpw, bpw)],
                            gh_h.at[pl.ds(k * B + base, bpw)])
            pltpu.sync_copy(trow_v.at[pl.ds(k * bpw, bpw)],
                            gt_h.at[pl.ds(k * B + base, bpw)])

    return sc_gather


def _prep_body(l1e_ref, l2e_ref, l1r_ref, l2r_ref, cee_ref, cer_ref):
    for s in range(S1):
        cee_ref[s * W2:(s + 1) * W2, :] = jnp.dot(
            l2e_ref[...], l1e_ref[s], preferred_element_type=jnp.float32)
        cer_ref[s * W2:(s + 1) * W2, :] = jnp.dot(
            l2r_ref[...], l1r_ref[s], preferred_element_type=jnp.float32)


def _dot_t(a, b):
    return lax.dot_general(a, b, (((1,), (1,)), ((), ())),
                           preferred_element_type=jnp.float32)


def _main_body(timeT_ref, es_ref, ep_ref, eo_ref, gh_ref, gt_ref,
               cee_ref, cer_ref,
               s_out, p_out, o_out, h_out, t_out):
    cee = cee_ref[...]
    for k in range(NSEL):
        h_out[k] = _dot_t(cee, gh_ref[k])
        t_out[k] = _dot_t(cee, gt_ref[k])
    tm = timeT_ref[...]

    def timesel(full):
        acc = tm[0:1, :] * full[0:W2, :]
        for s in range(1, S1):
            acc = acc + tm[s:s + 1, :] * full[s * W2:(s + 1) * W2, :]
        return acc

    s_out[...] = timesel(_dot_t(cee, es_ref[...]))
    o_out[...] = timesel(_dot_t(cee, eo_ref[...]))
    p_out[...] = timesel(_dot_t(cer_ref[...], ep_ref[...]))


def kernel(spo, time, nh, nt, entity_embedding, relation_embedding,
           e_layer1, e_layer2, r_layer1, r_layer2):
    B = spo.shape[0]
    ne = entity_embedding.shape[0]
    nr = relation_embedding.shape[0]

    ent16 = entity_embedding.astype(jnp.bfloat16)
    rel16 = relation_embedding.astype(jnp.bfloat16)
    timeT = time.astype(jnp.float32).T
    nhT = nh.astype(jnp.int32).T
    ntT = nt.astype(jnp.int32).T
    spoT = spo.astype(jnp.int32).T

    es, ep, eo, gh, gt = _build_sc_gather(B, ne, nr)(
        ent16, rel16, timeT, nhT, ntT, spoT)
    gh = gh.reshape(NSEL, B, D)
    gt = gt.reshape(NSEL, B, D)

    cee, cer = pl.pallas_call(
        _prep_body,
        out_shape=[
            jax.ShapeDtypeStruct((S1 * W2, D), jnp.float32),
            jax.ShapeDtypeStruct((S1 * W2, D), jnp.float32),
        ],
    )(e_layer1.reshape(S1, D, D), e_layer2,
      r_layer1.reshape(S1, D, D), r_layer2)
    cee = cee.astype(jnp.bfloat16)
    cer = cer.astype(jnp.bfloat16)

    nblk = B // BLKB
    s_o, p_o, o_o, h_o, t_o = pl.pallas_call(
        _main_body,
        grid=(nblk,),
        in_specs=[
            pl.BlockSpec((16, BLKB), lambda i: (0, i)),
            pl.BlockSpec((BLKB, D), lambda i: (i, 0)),
            pl.BlockSpec((BLKB, D), lambda i: (i, 0)),
            pl.BlockSpec((BLKB, D), lambda i: (i, 0)),
            pl.BlockSpec((NSEL, BLKB, D), lambda i: (0, i, 0)),
            pl.BlockSpec((NSEL, BLKB, D), lambda i: (0, i, 0)),
            pl.BlockSpec((S1 * W2, D), lambda i: (0, 0)),
            pl.BlockSpec((S1 * W2, D), lambda i: (0, 0)),
        ],
        out_specs=[
            pl.BlockSpec((W2, BLKB), lambda i: (0, i)),
            pl.BlockSpec((W2, BLKB), lambda i: (0, i)),
            pl.BlockSpec((W2, BLKB), lambda i: (0, i)),
            pl.BlockSpec((NSEL, S1 * W2, BLKB), lambda i: (0, 0, i)),
            pl.BlockSpec((NSEL, S1 * W2, BLKB), lambda i: (0, 0, i)),
        ],
        out_shape=[
            jax.ShapeDtypeStruct((W2, B), jnp.float32),
            jax.ShapeDtypeStruct((W2, B), jnp.float32),
            jax.ShapeDtypeStruct((W2, B), jnp.float32),
            jax.ShapeDtypeStruct((NSEL, S1 * W2, B), jnp.float32),
            jax.ShapeDtypeStruct((NSEL, S1 * W2, B), jnp.float32),
        ],
    )(timeT, es, ep, eo, gh, gt, cee, cer)

    def back(x, m):
        return jnp.transpose(x.reshape(-1, B), (1, 0)).reshape(B, S2, m, D)

    return (back(s_o, 1), back(p_o, 1), back(o_o, 1),
            back(h_o, 4 * NSEL), back(t_o, 4 * NSEL))
